# baseline (device time: 312387 ns/iter reference)
import jax
import jax.numpy as jnp
from jax import lax
from jax.experimental import pallas as pl
from jax.experimental.pallas import tpu as pltpu

N_Y = 4


def kernel(O, Wo):
    B, S, Hl, D = O.shape
    K = Hl * D
    N = Wo.shape[1]
    S_out = S // N_Y
    M = B * S_out

    O2 = O.reshape(B, S, K)

    def body(o_ref, w_ref, out_ref, comm_ref, acc_ref, send_sems, recv_sems):
        my_x = lax.axis_index("x")
        my_y = lax.axis_index("y")
        my_z = lax.axis_index("z")
        right = lax.rem(my_y + 1, N_Y)
        left = lax.rem(my_y + N_Y - 1, N_Y)

        barrier_sem = pltpu.get_barrier_semaphore()
        for nbr in (left, right):
            pl.semaphore_signal(
                barrier_sem, inc=1,
                device_id=(my_x, nbr, my_z),
                device_id_type=pl.DeviceIdType.MESH,
            )
        pl.semaphore_wait(barrier_sem, 2)

        def chunk_partial(c):
            o = o_ref[:, pl.ds(c * S_out, S_out), :]
            o2 = o.reshape(M, K)
            return jnp.dot(o2, w_ref[...], preferred_element_type=jnp.float32)

        acc_ref[...] = chunk_partial(lax.rem(my_y + N_Y - 1, N_Y))

        for s in range(N_Y - 1):
            rdma = pltpu.make_async_remote_copy(
                src_ref=acc_ref,
                dst_ref=comm_ref.at[s],
                send_sem=send_sems.at[s],
                recv_sem=recv_sems.at[s],
                device_id=(my_x, right, my_z),
                device_id_type=pl.DeviceIdType.MESH,
            )
            rdma.start()
            rdma.wait()
            c = lax.rem(my_y + 2 * N_Y - 2 - s, N_Y)
            acc_ref[...] = chunk_partial(c) + comm_ref[s]

        out_ref[...] = acc_ref[...].reshape(B, S_out, N)

    return pl.pallas_call(
        body,
        out_shape=jax.ShapeDtypeStruct((B, S_out, N), jnp.float32),
        in_specs=[
            pl.BlockSpec(memory_space=pltpu.VMEM),
            pl.BlockSpec(memory_space=pltpu.VMEM),
        ],
        out_specs=pl.BlockSpec(memory_space=pltpu.VMEM),
        scratch_shapes=[
            pltpu.VMEM((N_Y - 1, M, N), jnp.float32),
            pltpu.VMEM((M, N), jnp.float32),
            pltpu.SemaphoreType.DMA((N_Y - 1,)),
            pltpu.SemaphoreType.DMA((N_Y - 1,)),
        ],
        compiler_params=pltpu.CompilerParams(
            collective_id=0,
            vmem_limit_bytes=64 * 1024 * 1024,
        ),
    )(O2, Wo)


# device time: 298983 ns/iter; 1.0448x vs baseline; 1.0448x over previous
import jax
import jax.numpy as jnp
from jax import lax
from jax.experimental import pallas as pl
from jax.experimental.pallas import tpu as pltpu

N_Y = 4


def kernel(O, Wo):
    B, S, Hl, D = O.shape
    K = Hl * D
    N = Wo.shape[1]
    S_out = S // N_Y
    M = B * S_out

    O2 = O.reshape(B, S, K)

    def body(o_ref, w_ref, out_ref, comm_ref, acc_ref, send_sems, recv_sems):
        my_x = lax.axis_index("x")
        my_y = lax.axis_index("y")
        my_z = lax.axis_index("z")
        right = lax.rem(my_y + 1, N_Y)
        left = lax.rem(my_y + N_Y - 1, N_Y)

        barrier_sem = pltpu.get_barrier_semaphore()
        for nbr in (left, right):
            pl.semaphore_signal(
                barrier_sem, inc=1,
                device_id=(my_x, nbr, my_z),
                device_id_type=pl.DeviceIdType.MESH,
            )
        pl.semaphore_wait(barrier_sem, 2)

        def chunk_partial(c):
            o = o_ref[:, pl.ds(c * S_out, S_out), :]
            o2 = o.reshape(M, K)
            return jnp.dot(o2, w_ref[...], preferred_element_type=jnp.float32)

        acc_ref[...] = chunk_partial(lax.rem(my_y + N_Y - 1, N_Y))

        for s in range(N_Y - 1):
            rdma = pltpu.make_async_remote_copy(
                src_ref=acc_ref,
                dst_ref=comm_ref.at[s],
                send_sem=send_sems.at[s],
                recv_sem=recv_sems.at[s],
                device_id=(my_x, right, my_z),
                device_id_type=pl.DeviceIdType.MESH,
            )
            rdma.start()
            rdma.wait()
            c = lax.rem(my_y + 2 * N_Y - 2 - s, N_Y)
            acc_ref[...] = chunk_partial(c) + comm_ref[s]

        out_ref[...] = acc_ref[...].reshape(B, S_out, N)

    return pl.pallas_call(
        body,
        out_shape=jax.ShapeDtypeStruct((B, S_out, N), jnp.float32),
        in_specs=[
            pl.BlockSpec(memory_space=pltpu.VMEM),
            pl.BlockSpec(memory_space=pltpu.VMEM),
        ],
        out_specs=pl.BlockSpec(memory_space=pltpu.VMEM),
        scratch_shapes=[
            pltpu.VMEM((N_Y - 1, M, N), jnp.float32),
            pltpu.VMEM((M, N), jnp.float32),
            pltpu.SemaphoreType.DMA((N_Y - 1,)),
            pltpu.SemaphoreType.DMA((N_Y - 1,)),
        ],
        compiler_params=pltpu.CompilerParams(collective_id=0),
    )(O2, Wo)


# device time: 288513 ns/iter; 1.0827x vs baseline; 1.0363x over previous
import jax
import jax.numpy as jnp
from jax import lax
from jax.experimental import pallas as pl
from jax.experimental.pallas import tpu as pltpu

N_Y = 4
Q = 2


def kernel(O, Wo):
    B, S, Hl, D = O.shape
    K = Hl * D
    N = Wo.shape[1]
    S_out = S // N_Y
    M = B * S_out

    O2 = O.reshape(B, S, K)

    def body(
        o_ref, w_ref, out_ref,
        comm_ref, acc_ref, send_sems, recv_sems, credit_sems,
    ):
        my_x = lax.axis_index("x")
        my_y = lax.axis_index("y")
        my_z = lax.axis_index("z")
        right = lax.rem(my_y + 1, N_Y)
        left = lax.rem(my_y + N_Y - 1, N_Y)

        barrier_sem = pltpu.get_barrier_semaphore()
        for nbr in (left, right):
            pl.semaphore_signal(
                barrier_sem, inc=1,
                device_id=(my_x, nbr, my_z),
                device_id_type=pl.DeviceIdType.MESH,
            )
        pl.semaphore_wait(barrier_sem, 2)

        Mq = M // Q
        BQ = B // Q

        def sub_partial(c, q):
            o = o_ref[pl.ds(q * BQ, BQ), pl.ds(c * S_out, S_out), :]
            o2 = o.reshape(Mq, K)
            return jnp.dot(o2, w_ref[...], preferred_element_type=jnp.float32)

        def make_rdma(s, q):
            return pltpu.make_async_remote_copy(
                src_ref=acc_ref.at[pl.ds(q * Mq, Mq)],
                dst_ref=comm_ref.at[s % 2, pl.ds(q * Mq, Mq)],
                send_sem=send_sems.at[s % 2, q],
                recv_sem=recv_sems.at[s % 2, q],
                device_id=(my_x, right, my_z),
                device_id_type=pl.DeviceIdType.MESH,
            )

        c0 = lax.rem(my_y + N_Y - 1, N_Y)
        for q in range(Q):
            acc_ref[pl.ds(q * Mq, Mq)] = sub_partial(c0, q)
            make_rdma(0, q).start()

        for s in range(N_Y - 1):
            c = lax.rem(my_y + 2 * N_Y - 2 - s, N_Y)
            for q in range(Q):
                make_rdma(s, q).wait()
                acc_ref[pl.ds(q * Mq, Mq)] = (
                    sub_partial(c, q) + comm_ref[s % 2, pl.ds(q * Mq, Mq)]
                )
                if s == 0:
                    pl.semaphore_signal(
                        credit_sems.at[q], inc=1,
                        device_id=(my_x, left, my_z),
                        device_id_type=pl.DeviceIdType.MESH,
                    )
                if s < N_Y - 2:
                    if s + 1 == 2:
                        pl.semaphore_wait(credit_sems.at[q], 1)
                    make_rdma(s + 1, q).start()

        out_ref[...] = acc_ref[...].reshape(B, S_out, N)

    return pl.pallas_call(
        body,
        out_shape=jax.ShapeDtypeStruct((B, S_out, N), jnp.float32),
        in_specs=[
            pl.BlockSpec(memory_space=pltpu.VMEM),
            pl.BlockSpec(memory_space=pltpu.VMEM),
        ],
        out_specs=pl.BlockSpec(memory_space=pltpu.VMEM),
        scratch_shapes=[
            pltpu.VMEM((2, M, N), jnp.float32),
            pltpu.VMEM((M, N), jnp.float32),
            pltpu.SemaphoreType.DMA((2, Q)),
            pltpu.SemaphoreType.DMA((2, Q)),
            pltpu.SemaphoreType.REGULAR((Q,)),
        ],
        compiler_params=pltpu.CompilerParams(collective_id=0),
    )(O2, Wo)


# device time: 287622 ns/iter; 1.0861x vs baseline; 1.0031x over previous
import jax
import jax.numpy as jnp
from jax import lax
from jax.experimental import pallas as pl
from jax.experimental.pallas import tpu as pltpu

N_Y = 4
Q = 4


def kernel(O, Wo):
    B, S, Hl, D = O.shape
    K = Hl * D
    N = Wo.shape[1]
    S_out = S // N_Y
    M = B * S_out

    O2 = O.reshape(B, S, K)

    def body(
        o_ref, w_ref, out_ref,
        comm_ref, acc_ref, send_sems, recv_sems, credit_sems,
    ):
        my_x = lax.axis_index("x")
        my_y = lax.axis_index("y")
        my_z = lax.axis_index("z")
        right = lax.rem(my_y + 1, N_Y)
        left = lax.rem(my_y + N_Y - 1, N_Y)

        barrier_sem = pltpu.get_barrier_semaphore()
        for nbr in (left, right):
            pl.semaphore_signal(
                barrier_sem, inc=1,
                device_id=(my_x, nbr, my_z),
                device_id_type=pl.DeviceIdType.MESH,
            )
        pl.semaphore_wait(barrier_sem, 2)

        Mq = M // Q
        BQ = B // Q

        def sub_partial(c, q):
            o = o_ref[pl.ds(q * BQ, BQ), pl.ds(c * S_out, S_out), :]
            o2 = o.reshape(Mq, K)
            return jnp.dot(o2, w_ref[...], preferred_element_type=jnp.float32)

        def make_rdma(s, q):
            return pltpu.make_async_remote_copy(
                src_ref=acc_ref.at[pl.ds(q * Mq, Mq)],
                dst_ref=comm_ref.at[s % 2, pl.ds(q * Mq, Mq)],
                send_sem=send_sems.at[s % 2, q],
                recv_sem=recv_sems.at[s % 2, q],
                device_id=(my_x, right, my_z),
                device_id_type=pl.DeviceIdType.MESH,
            )

        c0 = lax.rem(my_y + N_Y - 1, N_Y)
        for q in range(Q):
            acc_ref[pl.ds(q * Mq, Mq)] = sub_partial(c0, q)
            make_rdma(0, q).start()

        for s in range(N_Y - 1):
            c = lax.rem(my_y + 2 * N_Y - 2 - s, N_Y)
            for q in range(Q):
                make_rdma(s, q).wait()
                acc_ref[pl.ds(q * Mq, Mq)] = (
                    sub_partial(c, q) + comm_ref[s % 2, pl.ds(q * Mq, Mq)]
                )
                if s == 0:
                    pl.semaphore_signal(
                        credit_sems.at[q], inc=1,
                        device_id=(my_x, left, my_z),
                        device_id_type=pl.DeviceIdType.MESH,
                    )
                if s < N_Y - 2:
                    if s + 1 == 2:
                        pl.semaphore_wait(credit_sems.at[q], 1)
                    make_rdma(s + 1, q).start()

        out_ref[...] = acc_ref[...].reshape(B, S_out, N)

    return pl.pallas_call(
        body,
        out_shape=jax.ShapeDtypeStruct((B, S_out, N), jnp.float32),
        in_specs=[
            pl.BlockSpec(memory_space=pltpu.VMEM),
            pl.BlockSpec(memory_space=pltpu.VMEM),
        ],
        out_specs=pl.BlockSpec(memory_space=pltpu.VMEM),
        scratch_shapes=[
            pltpu.VMEM((2, M, N), jnp.float32),
            pltpu.VMEM((M, N), jnp.float32),
            pltpu.SemaphoreType.DMA((2, Q)),
            pltpu.SemaphoreType.DMA((2, Q)),
            pltpu.SemaphoreType.REGULAR((Q,)),
        ],
        compiler_params=pltpu.CompilerParams(collective_id=0),
    )(O2, Wo)
